# baseline (device time: 78107 ns/iter reference)
import jax
import jax.numpy as jnp
from jax import lax
from jax.experimental import pallas as pl
from jax.experimental.pallas import tpu as pltpu

T = 2048
D = 1024
TH = T // 2
SIZES = (64, 128, 160, 160, 160, 160, 128, 64)
C = len(SIZES)
OFFS = tuple(sum(SIZES[:c]) for c in range(C))
assert sum(SIZES) == TH


def kernel(ids, E):
    v_local = E.shape[0]
    my_x = lax.axis_index("x")
    my_y = lax.axis_index("y")

    ids_half = lax.dynamic_slice(ids, (my_x * TH,), (TH,))
    local = ids_half - my_y * v_local
    in_shard = (local >= 0) & (local < v_local)
    safe = jnp.clip(local, 0, v_local - 1).astype(jnp.int32)
    maskf = in_shard.astype(jnp.float32).reshape(TH, 1)

    def body(ids_ref, mask_ref, e_ref, out_ref, *scratch):
        gath = scratch[0:C]
        comm = scratch[C:2 * C]
        summ = scratch[2 * C:3 * C]
        other = scratch[3 * C:4 * C]
        row_sem, send_a, recv_a, send_b, recv_b = scratch[4 * C:]

        x = lax.axis_index("x")
        y = lax.axis_index("y")
        y_nbr = (x, 1 - y)
        x_nbr = (1 - x, y)

        barrier = pltpu.get_barrier_semaphore()
        for nbr in (y_nbr, x_nbr):
            pl.semaphore_signal(
                barrier, inc=1, device_id=nbr,
                device_id_type=pl.DeviceIdType.MESH,
            )
        pl.semaphore_wait(barrier, 2)

        def issue_chunk(c):
            def one(k, _):
                pltpu.make_async_copy(
                    e_ref.at[pl.ds(ids_ref[OFFS[c] + k], 1), :],
                    gath[c].at[pl.ds(k, 1), :],
                    row_sem.at[c],
                ).start()
                return 0

            lax.fori_loop(0, SIZES[c], one, 0)

        def wait_chunk(c):
            pltpu.make_async_copy(
                e_ref.at[pl.ds(0, SIZES[c]), :],
                gath[c],
                row_sem.at[c],
            ).wait()

        def rdma_a(c):
            return pltpu.make_async_remote_copy(
                src_ref=gath[c],
                dst_ref=comm[c],
                send_sem=send_a.at[c],
                recv_sem=recv_a.at[c],
                device_id=y_nbr,
                device_id_type=pl.DeviceIdType.MESH,
            )

        def rdma_b(c):
            return pltpu.make_async_remote_copy(
                src_ref=summ[c],
                dst_ref=other[c],
                send_sem=send_b.at[c],
                recv_sem=recv_b.at[c],
                device_id=x_nbr,
                device_id_type=pl.DeviceIdType.MESH,
            )

        issue_chunk(0)
        issue_chunk(1)
        for c in range(C):
            if c + 2 < C:
                issue_chunk(c + 2)
            wait_chunk(c)
            rdma_a(c).start()

        for c in range(C):
            rdma_a(c).wait_recv()
            summ[c][:, :] = jnp.where(
                mask_ref[pl.ds(OFFS[c], SIZES[c]), :] > 0,
                gath[c][:, :],
                comm[c][:, :],
            )
            rdma_b(c).start()
            out_ref[pl.ds(x * TH + OFFS[c], SIZES[c]), :] = summ[c][:, :]

        for c in range(C):
            rdma_a(c).wait_send()
            rdma_b(c).wait()
            out_ref[pl.ds((1 - x) * TH + OFFS[c], SIZES[c]), :] = other[c][:, :]

    return pl.pallas_call(
        body,
        out_shape=jax.ShapeDtypeStruct((T, D), jnp.float32),
        in_specs=[
            pl.BlockSpec(memory_space=pltpu.SMEM),
            pl.BlockSpec(memory_space=pltpu.VMEM),
            pl.BlockSpec(memory_space=pl.ANY),
        ],
        out_specs=pl.BlockSpec(memory_space=pltpu.VMEM),
        scratch_shapes=(
            [pltpu.VMEM((s, D), jnp.float32) for s in SIZES]
            + [pltpu.VMEM((s, D), jnp.float32) for s in SIZES]
            + [pltpu.VMEM((s, D), jnp.float32) for s in SIZES]
            + [pltpu.VMEM((s, D), jnp.float32) for s in SIZES]
            + [
                pltpu.SemaphoreType.DMA((C,)),
                pltpu.SemaphoreType.DMA((C,)),
                pltpu.SemaphoreType.DMA((C,)),
                pltpu.SemaphoreType.DMA((C,)),
                pltpu.SemaphoreType.DMA((C,)),
            ]
        ),
        compiler_params=pltpu.CompilerParams(collective_id=0),
    )(safe, maskf, E)
